# Initial kernel scaffold; baseline (speedup 1.0000x reference)
import jax
import jax.numpy as jnp
from jax.experimental import pallas as pl


def kernel(hidden_states, router_w, gate_w, up_w, down_w):
    raise NotImplementedError("write your pallas kernel here")


# speedup vs baseline: 1.8059x; 1.8059x over previous
import functools

import jax
import jax.numpy as jnp
from jax import lax
from jax.experimental import pallas as pl
from jax.experimental.pallas import tpu as pltpu
from jax.experimental.pallas import tpu_sc as plsc

NUM_EXPERTS = 64
TOP_K = 2
HIDDEN = 768
INTER = 1536
T = 2048
CAP = 64
A = T * TOP_K
I_BLK = 768
N_IBLK = INTER // I_BLK

SC_NC = 2
SC_NS = 16
SC_NW = SC_NC * SC_NS
ROWS_W = A // SC_NW
LANES = 16


def _shift_down(c, sh):
    return jnp.concatenate(
        [jnp.zeros((sh, NUM_EXPERTS), jnp.float32), c[: T - sh]], axis=0)


def _router_body(x_ref, rw_ref, logits_ref, dsts_ref, dstg_ref, wk_ref):
    x = x_ref[...]
    logits = jnp.dot(x, rw_ref[...], preferred_element_type=jnp.float32)
    logits_ref[...] = logits
    m = jnp.max(logits, axis=1, keepdims=True)
    p = jnp.exp(logits - m)
    aff = p / jnp.sum(p, axis=1, keepdims=True)
    lane = lax.broadcasted_iota(jnp.int32, (T, NUM_EXPERTS), 1)
    v1 = jnp.max(aff, axis=1, keepdims=True)
    e1 = jnp.min(jnp.where(aff == v1, lane, NUM_EXPERTS), axis=1, keepdims=True)
    aff2 = jnp.where(lane == e1, -1.0, aff)
    v2 = jnp.max(aff2, axis=1, keepdims=True)
    e2 = jnp.min(jnp.where(aff2 == v2, lane, NUM_EXPERTS), axis=1, keepdims=True)
    tot = v1 + v2
    w1 = v1 / tot
    w2 = v2 / tot
    oh1 = (lane == e1).astype(jnp.float32)
    oh2 = (lane == e2).astype(jnp.float32)
    comb = oh1 + oh2
    c = comb
    sh = 1
    while sh < T:
        c = c + _shift_down(c, sh)
        sh *= 2
    excl = c - comb
    pos1 = jnp.sum(oh1 * excl, axis=1, keepdims=True).astype(jnp.int32)
    pos2 = jnp.sum(oh2 * excl, axis=1, keepdims=True).astype(jnp.int32)
    keep1 = pos1 < CAP
    keep2 = pos2 < CAP
    slot1 = e1 * CAP + jnp.minimum(pos1, CAP - 1)
    slot2 = e2 * CAP + jnp.minimum(pos2, CAP - 1)
    row = lax.broadcasted_iota(jnp.int32, (T, 1), 0)
    dump = A + (row & (LANES - 1))
    dsts_ref[...] = jnp.concatenate(
        [jnp.where(keep1, slot1, dump), jnp.where(keep2, slot2, dump)], axis=1)
    dstg_ref[...] = jnp.concatenate([slot1, slot2], axis=1)
    wk_ref[...] = jnp.concatenate(
        [jnp.where(keep1, w1, 0.0), jnp.where(keep2, w2, 0.0)], axis=1)


def _router(x, router_w):
    return pl.pallas_call(
        _router_body,
        out_shape=(
            jax.ShapeDtypeStruct((T, NUM_EXPERTS), jnp.float32),
            jax.ShapeDtypeStruct((T, TOP_K), jnp.int32),
            jax.ShapeDtypeStruct((T, TOP_K), jnp.int32),
            jax.ShapeDtypeStruct((T, TOP_K), jnp.float32),
        ),
    )(x, router_w)


def _sc_dispatch_body(x_hbm, dsts_hbm, disp_hbm, dstv, srcv, rows, sem):
    wid = lax.axis_index("s") * SC_NC + lax.axis_index("c")
    base = wid * ROWS_W
    pltpu.sync_copy(dsts_hbm, dstv)
    lanes = lax.iota(jnp.int32, 16)

    def init_body(j, carry):
        srcv[pl.ds(j * 16, 16)] = jnp.zeros((16,), jnp.int32)
        return carry

    lax.fori_loop(0, ROWS_W // 16, init_body, 0)

    def scan_body(cidx, carry):
        idx = dstv[pl.ds(cidx * 16, 16)]
        off = jnp.where(cidx >= T // 16, T, 0)
        tok = cidx * 16 - off + lanes
        loc = idx - base
        msk = (loc >= 0) & (loc < ROWS_W)
        locc = jnp.clip(loc, 0, ROWS_W - 1)
        plsc.store_scatter(srcv, [locc], tok, mask=msk)
        return carry

    lax.fori_loop(0, A // 16, scan_body, 0)
    pltpu.async_copy(x_hbm.at[srcv], rows, sem).wait()
    pltpu.sync_copy(rows, disp_hbm.at[pl.ds(base, ROWS_W)])


def _expert_body(disp_ref, gw_ref, uw_ref, dw_ref, out_ref):
    i = pl.program_id(1)
    d = disp_ref[...]
    g = jnp.dot(d, gw_ref[0], preferred_element_type=jnp.float32)
    u = jnp.dot(d, uw_ref[0], preferred_element_type=jnp.float32)
    h = g * (1.0 / (1.0 + jnp.exp(-g))) * u
    partial = jnp.dot(h, dw_ref[0], preferred_element_type=jnp.float32)

    @pl.when(i == 0)
    def _():
        out_ref[...] = partial

    @pl.when(i != 0)
    def _():
        out_ref[...] += partial


def _experts(disp, gate_w, up_w, down_w):
    return pl.pallas_call(
        _expert_body,
        grid=(NUM_EXPERTS, N_IBLK),
        in_specs=[
            pl.BlockSpec((CAP, HIDDEN), lambda e, i: (e, 0)),
            pl.BlockSpec((1, HIDDEN, I_BLK), lambda e, i: (e, 0, i)),
            pl.BlockSpec((1, HIDDEN, I_BLK), lambda e, i: (e, 0, i)),
            pl.BlockSpec((1, I_BLK, HIDDEN), lambda e, i: (e, i, 0)),
        ],
        out_specs=pl.BlockSpec((CAP, HIDDEN), lambda e, i: (e, 0)),
        out_shape=jax.ShapeDtypeStruct((A, HIDDEN), jnp.float32),
    )(disp, gate_w, up_w, down_w)


def _sc_gather_body(eo_hbm, dstg_hbm, g_hbm, idxv, rows, sem):
    wid = lax.axis_index("s") * SC_NC + lax.axis_index("c")
    base = wid * ROWS_W
    pltpu.sync_copy(dstg_hbm.at[pl.ds(base, ROWS_W)], idxv)
    pltpu.async_copy(eo_hbm.at[idxv], rows, sem).wait()
    pltpu.sync_copy(rows, g_hbm.at[pl.ds(base, ROWS_W)])


@functools.cache
def _sc_kernels():
    mesh = plsc.VectorSubcoreMesh(
        core_axis_name="c", subcore_axis_name="s",
        num_cores=SC_NC, num_subcores=SC_NS)
    params = pltpu.CompilerParams(needs_layout_passes=False)
    dispatch = pl.kernel(
        _sc_dispatch_body,
        mesh=mesh,
        compiler_params=params,
        out_type=jax.ShapeDtypeStruct((A, HIDDEN), jnp.float32),
        scratch_types=[
            pltpu.VMEM((A,), jnp.int32),
            pltpu.VMEM((ROWS_W,), jnp.int32),
            pltpu.VMEM((ROWS_W, HIDDEN), jnp.float32),
            pltpu.SemaphoreType.DMA,
        ],
    )
    gather = pl.kernel(
        _sc_gather_body,
        mesh=mesh,
        compiler_params=params,
        out_type=jax.ShapeDtypeStruct((A, HIDDEN), jnp.float32),
        scratch_types=[
            pltpu.VMEM((ROWS_W,), jnp.int32),
            pltpu.VMEM((ROWS_W, HIDDEN), jnp.float32),
            pltpu.SemaphoreType.DMA,
        ],
    )
    return dispatch, gather


def _combine_body(g0_ref, g1_ref, wk_ref, out_ref):
    wk = wk_ref[...]
    out_ref[...] = g0_ref[...] * wk[:, 0:1] + g1_ref[...] * wk[:, 1:2]


_CB = 256


def _combine(g, wk):
    return pl.pallas_call(
        _combine_body,
        grid=(T // _CB,),
        in_specs=[
            pl.BlockSpec((_CB, HIDDEN), lambda r: (r, 0)),
            pl.BlockSpec((_CB, HIDDEN), lambda r: (r + T // _CB, 0)),
            pl.BlockSpec((_CB, TOP_K), lambda r: (r, 0)),
        ],
        out_specs=pl.BlockSpec((_CB, HIDDEN), lambda r: (r, 0)),
        out_shape=jax.ShapeDtypeStruct((T, HIDDEN), jnp.float32),
    )(g, g, wk)


def kernel(hidden_states, router_w, gate_w, up_w, down_w):
    S_, B_, H = hidden_states.shape
    x = hidden_states.reshape(T, H)
    logits, dsts, dstg, wk = _router(x, router_w)
    dsts_flat = dsts.T.reshape(A)
    dstg_flat = dstg.T.reshape(A)
    sc_dispatch, sc_gather = _sc_kernels()
    disp = sc_dispatch(x, dsts_flat)
    eo = _experts(disp, gate_w, up_w, down_w)
    g = sc_gather(eo, dstg_flat)
    out = _combine(g, wk)
    return out.reshape(S_, B_, H), logits


# speedup vs baseline: 1.8435x; 1.0208x over previous
import functools

import jax
import jax.numpy as jnp
from jax import lax
from jax.experimental import pallas as pl
from jax.experimental.pallas import tpu as pltpu
from jax.experimental.pallas import tpu_sc as plsc

NUM_EXPERTS = 64
TOP_K = 2
HIDDEN = 768
INTER = 1536
T = 2048
CAP = 64
A = T * TOP_K
I_BLK = 768
N_IBLK = INTER // I_BLK
A_PAD = A + CAP

SC_NC = 2
SC_NS = 16
SC_NW = SC_NC * SC_NS
ROWS_W = A // SC_NW
LANES = 16


def _shift_down(c, sh):
    return jnp.concatenate(
        [jnp.zeros((sh, NUM_EXPERTS), jnp.float32), c[: T - sh]], axis=0)


def _router_body(x_ref, rw_ref, logits_ref, dsts_ref, dstg_ref, wk_ref):
    x = x_ref[...]
    logits = jnp.dot(x, rw_ref[...], preferred_element_type=jnp.float32)
    logits_ref[...] = logits
    m = jnp.max(logits, axis=1, keepdims=True)
    p = jnp.exp(logits - m)
    aff = p / jnp.sum(p, axis=1, keepdims=True)
    lane = lax.broadcasted_iota(jnp.int32, (T, NUM_EXPERTS), 1)
    v1 = jnp.max(aff, axis=1, keepdims=True)
    e1 = jnp.min(jnp.where(aff == v1, lane, NUM_EXPERTS), axis=1, keepdims=True)
    aff2 = jnp.where(lane == e1, -1.0, aff)
    v2 = jnp.max(aff2, axis=1, keepdims=True)
    e2 = jnp.min(jnp.where(aff2 == v2, lane, NUM_EXPERTS), axis=1, keepdims=True)
    tot = v1 + v2
    w1 = v1 / tot
    w2 = v2 / tot
    oh1 = (lane == e1).astype(jnp.float32)
    oh2 = (lane == e2).astype(jnp.float32)
    comb = oh1 + oh2
    c = comb
    sh = 1
    while sh < T:
        c = c + _shift_down(c, sh)
        sh *= 2
    excl = c - comb
    pos1 = jnp.sum(oh1 * excl, axis=1, keepdims=True).astype(jnp.int32)
    pos2 = jnp.sum(oh2 * excl, axis=1, keepdims=True).astype(jnp.int32)
    keep1 = pos1 < CAP
    keep2 = pos2 < CAP
    slot1 = e1 * CAP + jnp.minimum(pos1, CAP - 1)
    slot2 = e2 * CAP + jnp.minimum(pos2, CAP - 1)
    row = lax.broadcasted_iota(jnp.int32, (T, 1), 0)
    dump0 = A + ((2 * row) & (LANES - 1))
    dump1 = A + ((2 * row + 1) & (LANES - 1))
    dsts_ref[...] = jnp.concatenate(
        [jnp.where(keep1, slot1, dump0), jnp.where(keep2, slot2, dump1)], axis=1)
    dstg_ref[...] = jnp.concatenate([slot1, slot2], axis=1)
    wk_ref[...] = jnp.concatenate(
        [jnp.where(keep1, w1, 0.0), jnp.where(keep2, w2, 0.0)], axis=1)


def _router(x, router_w):
    return pl.pallas_call(
        _router_body,
        out_shape=(
            jax.ShapeDtypeStruct((T, NUM_EXPERTS), jnp.float32),
            jax.ShapeDtypeStruct((T, TOP_K), jnp.int32),
            jax.ShapeDtypeStruct((T, TOP_K), jnp.int32),
            jax.ShapeDtypeStruct((T, TOP_K), jnp.float32),
        ),
    )(x, router_w)


def _sc_dispatch_body(x_hbm, dsts_hbm, disp_hbm, tokv, dstv, rows, sem):
    wid = lax.axis_index("s") * SC_NC + lax.axis_index("c")
    base = wid * ROWS_W
    lanes = lax.iota(jnp.int32, 16)

    def mk_tok(c, carry):
        a = c * 16 + lanes
        tokv[pl.ds(c * 16, 16)] = wid * (ROWS_W // 2) + (a >> 1)
        return carry

    lax.fori_loop(0, ROWS_W // 16, mk_tok, 0)
    pltpu.sync_copy(dsts_hbm.at[pl.ds(base, ROWS_W)], dstv)
    pltpu.async_copy(x_hbm.at[tokv], rows, sem).wait()
    pltpu.async_copy(rows, disp_hbm.at[dstv], sem).wait()


def _expert_body(disp_ref, gw_ref, uw_ref, dw_ref, out_ref):
    i = pl.program_id(1)
    d = disp_ref[...]
    g = jnp.dot(d, gw_ref[0], preferred_element_type=jnp.float32)
    u = jnp.dot(d, uw_ref[0], preferred_element_type=jnp.float32)
    h = g * (1.0 / (1.0 + jnp.exp(-g))) * u
    partial = jnp.dot(h, dw_ref[0], preferred_element_type=jnp.float32)

    @pl.when(i == 0)
    def _():
        out_ref[...] = partial

    @pl.when(i != 0)
    def _():
        out_ref[...] += partial


def _experts(disp, gate_w, up_w, down_w):
    return pl.pallas_call(
        _expert_body,
        grid=(NUM_EXPERTS, N_IBLK),
        in_specs=[
            pl.BlockSpec((CAP, HIDDEN), lambda e, i: (e, 0)),
            pl.BlockSpec((1, HIDDEN, I_BLK), lambda e, i: (e, 0, i)),
            pl.BlockSpec((1, HIDDEN, I_BLK), lambda e, i: (e, 0, i)),
            pl.BlockSpec((1, I_BLK, HIDDEN), lambda e, i: (e, i, 0)),
        ],
        out_specs=pl.BlockSpec((CAP, HIDDEN), lambda e, i: (e, 0)),
        out_shape=jax.ShapeDtypeStruct((A, HIDDEN), jnp.float32),
    )(disp, gate_w, up_w, down_w)


def _sc_gather_body(eo_hbm, dstg_hbm, g_hbm, idxv, rows, sem):
    wid = lax.axis_index("s") * SC_NC + lax.axis_index("c")
    base = wid * ROWS_W
    pltpu.sync_copy(dstg_hbm.at[pl.ds(base, ROWS_W)], idxv)
    pltpu.async_copy(eo_hbm.at[idxv], rows, sem).wait()
    pltpu.sync_copy(rows, g_hbm.at[pl.ds(base, ROWS_W)])


@functools.cache
def _sc_kernels():
    mesh = plsc.VectorSubcoreMesh(
        core_axis_name="c", subcore_axis_name="s",
        num_cores=SC_NC, num_subcores=SC_NS)
    params = pltpu.CompilerParams(needs_layout_passes=False)
    dispatch = pl.kernel(
        _sc_dispatch_body,
        mesh=mesh,
        compiler_params=params,
        out_type=jax.ShapeDtypeStruct((A_PAD, HIDDEN), jnp.float32),
        scratch_types=[
            pltpu.VMEM((ROWS_W,), jnp.int32),
            pltpu.VMEM((ROWS_W,), jnp.int32),
            pltpu.VMEM((ROWS_W, HIDDEN), jnp.float32),
            pltpu.SemaphoreType.DMA,
        ],
    )
    gather = pl.kernel(
        _sc_gather_body,
        mesh=mesh,
        compiler_params=params,
        out_type=jax.ShapeDtypeStruct((A, HIDDEN), jnp.float32),
        scratch_types=[
            pltpu.VMEM((ROWS_W,), jnp.int32),
            pltpu.VMEM((ROWS_W, HIDDEN), jnp.float32),
            pltpu.SemaphoreType.DMA,
        ],
    )
    return dispatch, gather


def _combine_body(g_ref, wk_ref, out_ref):
    wk = wk_ref[...]
    out_ref[...] = (g_ref[:, :HIDDEN] * wk[:, 0:1]
                    + g_ref[:, HIDDEN:] * wk[:, 1:2])


_CB = 256


def _combine(g2, wk):
    return pl.pallas_call(
        _combine_body,
        grid=(T // _CB,),
        in_specs=[
            pl.BlockSpec((_CB, TOP_K * HIDDEN), lambda r: (r, 0)),
            pl.BlockSpec((_CB, TOP_K), lambda r: (r, 0)),
        ],
        out_specs=pl.BlockSpec((_CB, HIDDEN), lambda r: (r, 0)),
        out_shape=jax.ShapeDtypeStruct((T, HIDDEN), jnp.float32),
    )(g2, wk)


def kernel(hidden_states, router_w, gate_w, up_w, down_w):
    S_, B_, H = hidden_states.shape
    x = hidden_states.reshape(T, H)
    logits, dsts, dstg, wk = _router(x, router_w)
    dsts_flat = dsts.reshape(A)
    dstg_flat = dstg.reshape(A)
    sc_dispatch, sc_gather = _sc_kernels()
    disp = sc_dispatch(x, dsts_flat)
    eo = _experts(disp, gate_w, up_w, down_w)
    g = sc_gather(eo, dstg_flat)
    out = _combine(g.reshape(T, TOP_K * HIDDEN), wk)
    return out.reshape(S_, B_, H), logits


# speedup vs baseline: 1.8437x; 1.0001x over previous
import functools

import jax
import jax.numpy as jnp
from jax import lax
from jax.experimental import pallas as pl
from jax.experimental.pallas import tpu as pltpu
from jax.experimental.pallas import tpu_sc as plsc

NUM_EXPERTS = 64
TOP_K = 2
HIDDEN = 768
INTER = 1536
T = 2048
CAP = 64
A = T * TOP_K
I_BLK = 768
N_IBLK = INTER // I_BLK
A_PAD = A + CAP

SC_NC = 2
SC_NS = 16
SC_NW = SC_NC * SC_NS
ROWS_W = A // SC_NW
LANES = 16


def _shift_down(c, sh):
    return jnp.concatenate(
        [jnp.zeros((sh, NUM_EXPERTS), jnp.float32), c[: T - sh]], axis=0)


def _router_body(x_ref, rw_ref, logits_ref, dstw_ref, wk_ref):
    x = x_ref[...]
    logits = jnp.dot(x, rw_ref[...], preferred_element_type=jnp.float32)
    logits_ref[...] = logits
    m = jnp.max(logits, axis=1, keepdims=True)
    p = jnp.exp(logits - m)
    aff = p / jnp.sum(p, axis=1, keepdims=True)
    lane = lax.broadcasted_iota(jnp.int32, (T, NUM_EXPERTS), 1)
    v1 = jnp.max(aff, axis=1, keepdims=True)
    e1 = jnp.min(jnp.where(aff == v1, lane, NUM_EXPERTS), axis=1, keepdims=True)
    aff2 = jnp.where(lane == e1, -1.0, aff)
    v2 = jnp.max(aff2, axis=1, keepdims=True)
    e2 = jnp.min(jnp.where(aff2 == v2, lane, NUM_EXPERTS), axis=1, keepdims=True)
    tot = v1 + v2
    w1 = v1 / tot
    w2 = v2 / tot
    oh1 = (lane == e1).astype(jnp.float32)
    oh2 = (lane == e2).astype(jnp.float32)
    comb = oh1 + oh2
    c = comb
    sh = 1
    while sh < T:
        c = c + _shift_down(c, sh)
        sh *= 2
    excl = c - comb
    pos1 = jnp.sum(oh1 * excl, axis=1, keepdims=True).astype(jnp.int32)
    pos2 = jnp.sum(oh2 * excl, axis=1, keepdims=True).astype(jnp.int32)
    keep1 = pos1 < CAP
    keep2 = pos2 < CAP
    slot1 = e1 * CAP + jnp.minimum(pos1, CAP - 1)
    slot2 = e2 * CAP + jnp.minimum(pos2, CAP - 1)
    row = lax.broadcasted_iota(jnp.int32, (T, 1), 0)
    dump0 = A + ((2 * row) & (LANES - 1))
    dump1 = A + ((2 * row + 1) & (LANES - 1))
    l128 = lax.broadcasted_iota(jnp.int32, (T, 128), 1)
    dstw = jnp.where(l128 == 0, jnp.where(keep1, slot1, dump0),
           jnp.where(l128 == 1, jnp.where(keep2, slot2, dump1),
           jnp.where(l128 == 2, slot1,
           jnp.where(l128 == 3, slot2, 0))))
    dstw_ref[...] = dstw
    wk_ref[...] = jnp.concatenate(
        [jnp.where(keep1, w1, 0.0), jnp.where(keep2, w2, 0.0)], axis=1)


def _router(x, router_w):
    return pl.pallas_call(
        _router_body,
        out_shape=(
            jax.ShapeDtypeStruct((T, NUM_EXPERTS), jnp.float32),
            jax.ShapeDtypeStruct((T, 128), jnp.int32),
            jax.ShapeDtypeStruct((T, TOP_K), jnp.float32),
        ),
    )(x, router_w)


def _sc_dispatch_body(x_hbm, dstw_hbm, disp_hbm, wide, tokv, dstv, rows, sem):
    wid = lax.axis_index("s") * SC_NC + lax.axis_index("c")
    ntok = ROWS_W // 2
    lanes = lax.iota(jnp.int32, 16)
    pltpu.sync_copy(dstw_hbm.at[pl.ds(wid * ntok * 128, ntok * 128)], wide)

    def mk(c, carry):
        j = c * 16 + lanes
        dstv[pl.ds(c * 16, 16)] = plsc.load_gather(
            wide, [((j >> 1) << 7) + (j & 1)])
        tokv[pl.ds(c * 16, 16)] = wid * ntok + (j >> 1)
        return carry

    lax.fori_loop(0, ROWS_W // 16, mk, 0)
    pltpu.async_copy(x_hbm.at[tokv], rows, sem).wait()
    pltpu.async_copy(rows, disp_hbm.at[dstv], sem).wait()


def _expert_body(disp_ref, gw_ref, uw_ref, dw_ref, out_ref):
    i = pl.program_id(1)
    d = disp_ref[...]
    g = jnp.dot(d, gw_ref[0], preferred_element_type=jnp.float32)
    u = jnp.dot(d, uw_ref[0], preferred_element_type=jnp.float32)
    h = g * (1.0 / (1.0 + jnp.exp(-g))) * u
    partial = jnp.dot(h, dw_ref[0], preferred_element_type=jnp.float32)

    @pl.when(i == 0)
    def _():
        out_ref[...] = partial

    @pl.when(i != 0)
    def _():
        out_ref[...] += partial


def _experts(disp, gate_w, up_w, down_w):
    return pl.pallas_call(
        _expert_body,
        grid=(NUM_EXPERTS, N_IBLK),
        in_specs=[
            pl.BlockSpec((CAP, HIDDEN), lambda e, i: (e, 0)),
            pl.BlockSpec((1, HIDDEN, I_BLK), lambda e, i: (e, 0, i)),
            pl.BlockSpec((1, HIDDEN, I_BLK), lambda e, i: (e, 0, i)),
            pl.BlockSpec((1, I_BLK, HIDDEN), lambda e, i: (e, i, 0)),
        ],
        out_specs=pl.BlockSpec((CAP, HIDDEN), lambda e, i: (e, 0)),
        out_shape=jax.ShapeDtypeStruct((A, HIDDEN), jnp.float32),
    )(disp, gate_w, up_w, down_w)


def _sc_gather_body(eo_hbm, dstw_hbm, g_hbm, wide, idxv, rows, sem):
    wid = lax.axis_index("s") * SC_NC + lax.axis_index("c")
    ntok = ROWS_W // 2
    lanes = lax.iota(jnp.int32, 16)
    pltpu.sync_copy(dstw_hbm.at[pl.ds(wid * ntok * 128, ntok * 128)], wide)

    def mk(c, carry):
        j = c * 16 + lanes
        idxv[pl.ds(c * 16, 16)] = plsc.load_gather(
            wide, [((j >> 1) << 7) + 2 + (j & 1)])
        return carry

    lax.fori_loop(0, ROWS_W // 16, mk, 0)
    pltpu.async_copy(eo_hbm.at[idxv], rows, sem).wait()
    pltpu.sync_copy(rows, g_hbm.at[pl.ds(wid * ROWS_W, ROWS_W)])


@functools.cache
def _sc_kernels():
    mesh = plsc.VectorSubcoreMesh(
        core_axis_name="c", subcore_axis_name="s",
        num_cores=SC_NC, num_subcores=SC_NS)
    params = pltpu.CompilerParams(needs_layout_passes=False)
    dispatch = pl.kernel(
        _sc_dispatch_body,
        mesh=mesh,
        compiler_params=params,
        out_type=jax.ShapeDtypeStruct((A_PAD, HIDDEN), jnp.float32),
        scratch_types=[
            pltpu.VMEM((ROWS_W // 2 * 128,), jnp.int32),
            pltpu.VMEM((ROWS_W,), jnp.int32),
            pltpu.VMEM((ROWS_W,), jnp.int32),
            pltpu.VMEM((ROWS_W, HIDDEN), jnp.float32),
            pltpu.SemaphoreType.DMA,
        ],
    )
    gather = pl.kernel(
        _sc_gather_body,
        mesh=mesh,
        compiler_params=params,
        out_type=jax.ShapeDtypeStruct((A, HIDDEN), jnp.float32),
        scratch_types=[
            pltpu.VMEM((ROWS_W // 2 * 128,), jnp.int32),
            pltpu.VMEM((ROWS_W,), jnp.int32),
            pltpu.VMEM((ROWS_W, HIDDEN), jnp.float32),
            pltpu.SemaphoreType.DMA,
        ],
    )
    return dispatch, gather


def _combine_body(g_ref, wk_ref, out_ref):
    wk = wk_ref[...]
    out_ref[...] = (g_ref[:, :HIDDEN] * wk[:, 0:1]
                    + g_ref[:, HIDDEN:] * wk[:, 1:2])


_CB = 256


def _combine(g2, wk):
    return pl.pallas_call(
        _combine_body,
        grid=(T // _CB,),
        in_specs=[
            pl.BlockSpec((_CB, TOP_K * HIDDEN), lambda r: (r, 0)),
            pl.BlockSpec((_CB, TOP_K), lambda r: (r, 0)),
        ],
        out_specs=pl.BlockSpec((_CB, HIDDEN), lambda r: (r, 0)),
        out_shape=jax.ShapeDtypeStruct((T, HIDDEN), jnp.float32),
    )(g2, wk)


def kernel(hidden_states, router_w, gate_w, up_w, down_w):
    S_, B_, H = hidden_states.shape
    x = hidden_states.reshape(T, H)
    logits, dstw, wk = _router(x, router_w)
    dstw_flat = dstw.reshape(T * 128)
    sc_dispatch, sc_gather = _sc_kernels()
    disp = sc_dispatch(x, dstw_flat)
    eo = _experts(disp, gate_w, up_w, down_w)
    g = sc_gather(eo, dstw_flat)
    out = _combine(g.reshape(T, TOP_K * HIDDEN), wk)
    return out.reshape(S_, B_, H), logits


# speedup vs baseline: 1.8569x; 1.0072x over previous
import functools

import jax
import jax.numpy as jnp
from jax import lax
from jax.experimental import pallas as pl
from jax.experimental.pallas import tpu as pltpu
from jax.experimental.pallas import tpu_sc as plsc

NUM_EXPERTS = 64
TOP_K = 2
HIDDEN = 768
INTER = 1536
T = 2048
CAP = 64
A = T * TOP_K
I_BLK = 768
N_IBLK = INTER // I_BLK
A_PAD = A + CAP

SC_NC = 2
SC_NS = 16
SC_NW = SC_NC * SC_NS
ROWS_W = A // SC_NW
LANES = 16


def _shift_down(c, sh):
    return jnp.concatenate(
        [jnp.zeros((sh, NUM_EXPERTS), jnp.float32), c[: T - sh]], axis=0)


def _router_body(x_ref, rw_ref, logits_ref, dstw_ref, wk_ref):
    x = x_ref[...]
    logits = jnp.dot(x, rw_ref[...], preferred_element_type=jnp.float32)
    logits_ref[...] = logits
    m = jnp.max(logits, axis=1, keepdims=True)
    p = jnp.exp(logits - m)
    aff = p / jnp.sum(p, axis=1, keepdims=True)
    lane = lax.broadcasted_iota(jnp.int32, (T, NUM_EXPERTS), 1)
    v1 = jnp.max(aff, axis=1, keepdims=True)
    e1 = jnp.min(jnp.where(aff == v1, lane, NUM_EXPERTS), axis=1, keepdims=True)
    aff2 = jnp.where(lane == e1, -1.0, aff)
    v2 = jnp.max(aff2, axis=1, keepdims=True)
    e2 = jnp.min(jnp.where(aff2 == v2, lane, NUM_EXPERTS), axis=1, keepdims=True)
    tot = v1 + v2
    w1 = v1 / tot
    w2 = v2 / tot
    oh1 = (lane == e1).astype(jnp.float32)
    oh2 = (lane == e2).astype(jnp.float32)
    comb = oh1 + oh2
    c = comb
    sh = 1
    while sh < T:
        c = c + _shift_down(c, sh)
        sh *= 2
    excl = c - comb
    pos1 = jnp.sum(oh1 * excl, axis=1, keepdims=True).astype(jnp.int32)
    pos2 = jnp.sum(oh2 * excl, axis=1, keepdims=True).astype(jnp.int32)
    keep1 = pos1 < CAP
    keep2 = pos2 < CAP
    slot1 = e1 * CAP + jnp.minimum(pos1, CAP - 1)
    slot2 = e2 * CAP + jnp.minimum(pos2, CAP - 1)
    row = lax.broadcasted_iota(jnp.int32, (T, 1), 0)
    dump0 = A + ((2 * row) & (LANES - 1))
    dump1 = A + ((2 * row + 1) & (LANES - 1))
    l128 = lax.broadcasted_iota(jnp.int32, (T, 128), 1)
    dstw = jnp.where(l128 == 0, jnp.where(keep1, slot1, dump0),
           jnp.where(l128 == 1, jnp.where(keep2, slot2, dump1),
           jnp.where(l128 == 2, slot1,
           jnp.where(l128 == 3, slot2, 0))))
    dstw_ref[...] = dstw
    wk_ref[...] = jnp.concatenate(
        [jnp.where(keep1, w1, 0.0), jnp.where(keep2, w2, 0.0)], axis=1)


def _router(x, router_w):
    return pl.pallas_call(
        _router_body,
        out_shape=(
            jax.ShapeDtypeStruct((T, NUM_EXPERTS), jnp.float32),
            jax.ShapeDtypeStruct((T, 128), jnp.int32),
            jax.ShapeDtypeStruct((T, TOP_K), jnp.float32),
        ),
    )(x, router_w)


def _sc_dispatch_body(x_hbm, dstw_hbm, disp_hbm, wide, dst0, dst1, rows, sem):
    wid = lax.axis_index("s") * SC_NC + lax.axis_index("c")
    ntok = ROWS_W // 2
    lanes = lax.iota(jnp.int32, 16)
    pltpu.sync_copy(dstw_hbm.at[pl.ds(wid * ntok * 128, ntok * 128)], wide)
    cp = pltpu.async_copy(x_hbm.at[pl.ds(wid * ntok, ntok)], rows, sem)

    def mk(c, carry):
        t = c * 16 + lanes
        dst0[pl.ds(c * 16, 16)] = plsc.load_gather(wide, [t << 7])
        dst1[pl.ds(c * 16, 16)] = plsc.load_gather(wide, [(t << 7) + 1])
        return carry

    lax.fori_loop(0, ntok // 16, mk, 0)
    cp.wait()
    pltpu.async_copy(rows, disp_hbm.at[dst0], sem).wait()
    pltpu.async_copy(rows, disp_hbm.at[dst1], sem).wait()


def _expert_body(disp_ref, gw_ref, uw_ref, dw_ref, out_ref):
    i = pl.program_id(1)
    d = disp_ref[...]
    g = jnp.dot(d, gw_ref[0], preferred_element_type=jnp.float32)
    u = jnp.dot(d, uw_ref[0], preferred_element_type=jnp.float32)
    h = g * (1.0 / (1.0 + jnp.exp(-g))) * u
    partial = jnp.dot(h, dw_ref[0], preferred_element_type=jnp.float32)

    @pl.when(i == 0)
    def _():
        out_ref[...] = partial

    @pl.when(i != 0)
    def _():
        out_ref[...] += partial


def _experts(disp, gate_w, up_w, down_w):
    return pl.pallas_call(
        _expert_body,
        grid=(NUM_EXPERTS, N_IBLK),
        in_specs=[
            pl.BlockSpec((CAP, HIDDEN), lambda e, i: (e, 0)),
            pl.BlockSpec((1, HIDDEN, I_BLK), lambda e, i: (e, 0, i)),
            pl.BlockSpec((1, HIDDEN, I_BLK), lambda e, i: (e, 0, i)),
            pl.BlockSpec((1, I_BLK, HIDDEN), lambda e, i: (e, i, 0)),
        ],
        out_specs=pl.BlockSpec((CAP, HIDDEN), lambda e, i: (e, 0)),
        out_shape=jax.ShapeDtypeStruct((A, HIDDEN), jnp.float32),
    )(disp, gate_w, up_w, down_w)


def _sc_gather_body(eo_hbm, dstw_hbm, g_hbm, wide, idxv, rows, sem):
    wid = lax.axis_index("s") * SC_NC + lax.axis_index("c")
    ntok = ROWS_W // 2
    lanes = lax.iota(jnp.int32, 16)
    pltpu.sync_copy(dstw_hbm.at[pl.ds(wid * ntok * 128, ntok * 128)], wide)

    def mk(c, carry):
        j = c * 16 + lanes
        idxv[pl.ds(c * 16, 16)] = plsc.load_gather(
            wide, [((j >> 1) << 7) + 2 + (j & 1)])
        return carry

    lax.fori_loop(0, ROWS_W // 16, mk, 0)
    pltpu.async_copy(eo_hbm.at[idxv], rows, sem).wait()
    pltpu.sync_copy(rows, g_hbm.at[pl.ds(wid * ROWS_W, ROWS_W)])


@functools.cache
def _sc_kernels():
    mesh = plsc.VectorSubcoreMesh(
        core_axis_name="c", subcore_axis_name="s",
        num_cores=SC_NC, num_subcores=SC_NS)
    params = pltpu.CompilerParams(needs_layout_passes=False)
    dispatch = pl.kernel(
        _sc_dispatch_body,
        mesh=mesh,
        compiler_params=params,
        out_type=jax.ShapeDtypeStruct((A_PAD, HIDDEN), jnp.float32),
        scratch_types=[
            pltpu.VMEM((ROWS_W // 2 * 128,), jnp.int32),
            pltpu.VMEM((ROWS_W // 2,), jnp.int32),
            pltpu.VMEM((ROWS_W // 2,), jnp.int32),
            pltpu.VMEM((ROWS_W // 2, HIDDEN), jnp.float32),
            pltpu.SemaphoreType.DMA,
        ],
    )
    gather = pl.kernel(
        _sc_gather_body,
        mesh=mesh,
        compiler_params=params,
        out_type=jax.ShapeDtypeStruct((A, HIDDEN), jnp.float32),
        scratch_types=[
            pltpu.VMEM((ROWS_W // 2 * 128,), jnp.int32),
            pltpu.VMEM((ROWS_W,), jnp.int32),
            pltpu.VMEM((ROWS_W, HIDDEN), jnp.float32),
            pltpu.SemaphoreType.DMA,
        ],
    )
    return dispatch, gather


def _combine_body(g_ref, wk_ref, out_ref):
    wk = wk_ref[...]
    out_ref[...] = (g_ref[:, :HIDDEN] * wk[:, 0:1]
                    + g_ref[:, HIDDEN:] * wk[:, 1:2])


_CB = 256


def _combine(g2, wk):
    return pl.pallas_call(
        _combine_body,
        grid=(T // _CB,),
        in_specs=[
            pl.BlockSpec((_CB, TOP_K * HIDDEN), lambda r: (r, 0)),
            pl.BlockSpec((_CB, TOP_K), lambda r: (r, 0)),
        ],
        out_specs=pl.BlockSpec((_CB, HIDDEN), lambda r: (r, 0)),
        out_shape=jax.ShapeDtypeStruct((T, HIDDEN), jnp.float32),
    )(g2, wk)


def kernel(hidden_states, router_w, gate_w, up_w, down_w):
    S_, B_, H = hidden_states.shape
    x = hidden_states.reshape(T, H)
    logits, dstw, wk = _router(x, router_w)
    dstw_flat = dstw.reshape(T * 128)
    sc_dispatch, sc_gather = _sc_kernels()
    disp = sc_dispatch(x, dstw_flat)
    eo = _experts(disp, gate_w, up_w, down_w)
    g = sc_gather(eo, dstw_flat)
    out = _combine(g.reshape(T, TOP_K * HIDDEN), wk)
    return out.reshape(S_, B_, H), logits


# speedup vs baseline: 1.8646x; 1.0041x over previous
import functools

import jax
import jax.numpy as jnp
from jax import lax
from jax.experimental import pallas as pl
from jax.experimental.pallas import tpu as pltpu
from jax.experimental.pallas import tpu_sc as plsc

NUM_EXPERTS = 64
TOP_K = 2
HIDDEN = 768
INTER = 1536
T = 2048
CAP = 64
A = T * TOP_K
I_BLK = 768
N_IBLK = INTER // I_BLK
A_PAD = A + CAP

SC_NC = 2
SC_NS = 16
SC_NW = SC_NC * SC_NS
ROWS_W = A // SC_NW
LANES = 16


def _shift_down(c, sh):
    return jnp.concatenate(
        [jnp.zeros((sh, NUM_EXPERTS), jnp.float32), c[: T - sh]], axis=0)


def _router_body(x_ref, rw_ref, logits_ref, dstw_ref, wk_ref):
    x = x_ref[...]
    logits = jnp.dot(x, rw_ref[...], preferred_element_type=jnp.float32)
    logits_ref[...] = logits
    m = jnp.max(logits, axis=1, keepdims=True)
    p = jnp.exp(logits - m)
    aff = p / jnp.sum(p, axis=1, keepdims=True)
    lane = lax.broadcasted_iota(jnp.int32, (T, NUM_EXPERTS), 1)
    v1 = jnp.max(aff, axis=1, keepdims=True)
    e1 = jnp.min(jnp.where(aff == v1, lane, NUM_EXPERTS), axis=1, keepdims=True)
    aff2 = jnp.where(lane == e1, -1.0, aff)
    v2 = jnp.max(aff2, axis=1, keepdims=True)
    e2 = jnp.min(jnp.where(aff2 == v2, lane, NUM_EXPERTS), axis=1, keepdims=True)
    tot = v1 + v2
    w1 = v1 / tot
    w2 = v2 / tot
    oh1 = (lane == e1).astype(jnp.float32)
    oh2 = (lane == e2).astype(jnp.float32)
    comb = oh1 + oh2
    c = comb
    sh = 1
    while sh < T:
        c = c + _shift_down(c, sh)
        sh *= 2
    excl = c - comb
    pos1 = jnp.sum(oh1 * excl, axis=1, keepdims=True).astype(jnp.int32)
    pos2 = jnp.sum(oh2 * excl, axis=1, keepdims=True).astype(jnp.int32)
    keep1 = pos1 < CAP
    keep2 = pos2 < CAP
    slot1 = e1 * CAP + jnp.minimum(pos1, CAP - 1)
    slot2 = e2 * CAP + jnp.minimum(pos2, CAP - 1)
    row = lax.broadcasted_iota(jnp.int32, (T, 1), 0)
    dump0 = A + (row & (LANES - 1))
    dump1 = A + LANES + (row & (LANES - 1))
    l128 = lax.broadcasted_iota(jnp.int32, (T, 128), 1)
    dstw = jnp.where(l128 == 0, jnp.where(keep1, slot1, dump0),
           jnp.where(l128 == 1, jnp.where(keep2, slot2, dump1),
           jnp.where(l128 == 2, slot1,
           jnp.where(l128 == 3, slot2, 0))))
    dstw_ref[...] = dstw
    wk_ref[...] = jnp.concatenate(
        [jnp.where(keep1, w1, 0.0), jnp.where(keep2, w2, 0.0)], axis=1)


def _router(x, router_w):
    return pl.pallas_call(
        _router_body,
        out_shape=(
            jax.ShapeDtypeStruct((T, NUM_EXPERTS), jnp.float32),
            jax.ShapeDtypeStruct((T, 128), jnp.int32),
            jax.ShapeDtypeStruct((T, TOP_K), jnp.float32),
        ),
    )(x, router_w)


def _sc_dispatch_body(x_hbm, dstw_hbm, disp_hbm, wide, dst0, dst1, rows, sem):
    wid = lax.axis_index("s") * SC_NC + lax.axis_index("c")
    ntok = ROWS_W // 2
    lanes = lax.iota(jnp.int32, 16)
    pltpu.sync_copy(dstw_hbm.at[pl.ds(wid * ntok * 128, ntok * 128)], wide)
    cp = pltpu.async_copy(x_hbm.at[pl.ds(wid * ntok, ntok)], rows, sem)

    def mk(c, carry):
        t = c * 16 + lanes
        dst0[pl.ds(c * 16, 16)] = plsc.load_gather(wide, [t << 7])
        dst1[pl.ds(c * 16, 16)] = plsc.load_gather(wide, [(t << 7) + 1])
        return carry

    lax.fori_loop(0, ntok // 16, mk, 0)
    cp.wait()
    pltpu.async_copy(rows, disp_hbm.at[dst0], sem).wait()
    pltpu.async_copy(rows, disp_hbm.at[dst1], sem).wait()


def _expert_body(disp_ref, gw_ref, uw_ref, dw_ref, out_ref):
    i = pl.program_id(1)
    d = disp_ref[...]
    g = jnp.dot(d, gw_ref[0], preferred_element_type=jnp.float32)
    u = jnp.dot(d, uw_ref[0], preferred_element_type=jnp.float32)
    h = g * (1.0 / (1.0 + jnp.exp(-g))) * u
    partial = jnp.dot(h, dw_ref[0], preferred_element_type=jnp.float32)

    @pl.when(i == 0)
    def _():
        out_ref[...] = partial

    @pl.when(i != 0)
    def _():
        out_ref[...] += partial


def _experts(disp, gate_w, up_w, down_w):
    return pl.pallas_call(
        _expert_body,
        grid=(NUM_EXPERTS, N_IBLK),
        in_specs=[
            pl.BlockSpec((CAP, HIDDEN), lambda e, i: (e, 0)),
            pl.BlockSpec((1, HIDDEN, I_BLK), lambda e, i: (e, 0, i)),
            pl.BlockSpec((1, HIDDEN, I_BLK), lambda e, i: (e, 0, i)),
            pl.BlockSpec((1, I_BLK, HIDDEN), lambda e, i: (e, i, 0)),
        ],
        out_specs=pl.BlockSpec((CAP, HIDDEN), lambda e, i: (e, 0)),
        out_shape=jax.ShapeDtypeStruct((A, HIDDEN), jnp.float32),
    )(disp, gate_w, up_w, down_w)


def _sc_gather_body(eo_hbm, dstw_hbm, g_hbm, wide, idxv, rows, sem):
    wid = lax.axis_index("s") * SC_NC + lax.axis_index("c")
    ntok = ROWS_W // 2
    lanes = lax.iota(jnp.int32, 16)
    pltpu.sync_copy(dstw_hbm.at[pl.ds(wid * ntok * 128, ntok * 128)], wide)

    def mk(c, carry):
        j = c * 16 + lanes
        idxv[pl.ds(c * 16, 16)] = plsc.load_gather(
            wide, [((j >> 1) << 7) + 2 + (j & 1)])
        return carry

    lax.fori_loop(0, ROWS_W // 16, mk, 0)
    pltpu.async_copy(eo_hbm.at[idxv], rows, sem).wait()
    pltpu.sync_copy(rows, g_hbm.at[pl.ds(wid * ROWS_W, ROWS_W)])


@functools.cache
def _sc_kernels():
    mesh = plsc.VectorSubcoreMesh(
        core_axis_name="c", subcore_axis_name="s",
        num_cores=SC_NC, num_subcores=SC_NS)
    params = pltpu.CompilerParams(needs_layout_passes=False)
    dispatch = pl.kernel(
        _sc_dispatch_body,
        mesh=mesh,
        compiler_params=params,
        out_type=jax.ShapeDtypeStruct((A_PAD, HIDDEN), jnp.float32),
        scratch_types=[
            pltpu.VMEM((ROWS_W // 2 * 128,), jnp.int32),
            pltpu.VMEM((ROWS_W // 2,), jnp.int32),
            pltpu.VMEM((ROWS_W // 2,), jnp.int32),
            pltpu.VMEM((ROWS_W // 2, HIDDEN), jnp.float32),
            pltpu.SemaphoreType.DMA,
        ],
    )
    gather = pl.kernel(
        _sc_gather_body,
        mesh=mesh,
        compiler_params=params,
        out_type=jax.ShapeDtypeStruct((A, HIDDEN), jnp.float32),
        scratch_types=[
            pltpu.VMEM((ROWS_W // 2 * 128,), jnp.int32),
            pltpu.VMEM((ROWS_W,), jnp.int32),
            pltpu.VMEM((ROWS_W, HIDDEN), jnp.float32),
            pltpu.SemaphoreType.DMA,
        ],
    )
    return dispatch, gather


def _combine_body(g_ref, wk_ref, out_ref):
    wk = wk_ref[...]
    out_ref[...] = (g_ref[:, :HIDDEN] * wk[:, 0:1]
                    + g_ref[:, HIDDEN:] * wk[:, 1:2])


_CB = 256


def _combine(g2, wk):
    return pl.pallas_call(
        _combine_body,
        grid=(T // _CB,),
        in_specs=[
            pl.BlockSpec((_CB, TOP_K * HIDDEN), lambda r: (r, 0)),
            pl.BlockSpec((_CB, TOP_K), lambda r: (r, 0)),
        ],
        out_specs=pl.BlockSpec((_CB, HIDDEN), lambda r: (r, 0)),
        out_shape=jax.ShapeDtypeStruct((T, HIDDEN), jnp.float32),
    )(g2, wk)


def kernel(hidden_states, router_w, gate_w, up_w, down_w):
    S_, B_, H = hidden_states.shape
    x = hidden_states.reshape(T, H)
    logits, dstw, wk = _router(x, router_w)
    dstw_flat = dstw.reshape(T * 128)
    sc_dispatch, sc_gather = _sc_kernels()
    disp = sc_dispatch(x, dstw_flat)
    eo = _experts(disp, gate_w, up_w, down_w)
    g = sc_gather(eo, dstw_flat)
    out = _combine(g.reshape(T, TOP_K * HIDDEN), wk)
    return out.reshape(S_, B_, H), logits
